# Initial kernel scaffold; baseline (speedup 1.0000x reference)
import jax
import jax.numpy as jnp
from jax.experimental import pallas as pl


def kernel(x, adj, W1, b1, W2, b2):
    raise NotImplementedError("write your pallas kernel here")


# speedup vs baseline: 1.0144x; 1.0144x over previous
import functools

import jax
import jax.numpy as jnp
from jax.experimental import pallas as pl


def _gcn_layer_kernel(adj_ref, v_ref, w_ref, b_ref, out_ref, *, relu):
    a16 = adj_ref[...].astype(jnp.bfloat16)
    t = jnp.dot(a16, v_ref[...], preferred_element_type=jnp.float32)
    t = jnp.dot(t.astype(jnp.bfloat16), w_ref[...].astype(jnp.bfloat16),
                preferred_element_type=jnp.float32) + b_ref[...]
    if relu:
        t = jnp.maximum(t, 0.0)
    out_ref[...] = t.astype(out_ref.dtype)


def _gcn_layer(adj, v, w, b, *, relu, out_dtype, bm):
    n, k = adj.shape
    d = w.shape[1]
    grid = (n // bm,)
    return pl.pallas_call(
        functools.partial(_gcn_layer_kernel, relu=relu),
        grid=grid,
        in_specs=[
            pl.BlockSpec((bm, k), lambda i: (i, 0)),
            pl.BlockSpec((k, v.shape[1]), lambda i: (0, 0)),
            pl.BlockSpec(w.shape, lambda i: (0, 0)),
            pl.BlockSpec((1, d), lambda i: (0, 0)),
        ],
        out_specs=pl.BlockSpec((bm, d), lambda i: (i, 0)),
        out_shape=jax.ShapeDtypeStruct((n, d), out_dtype),
    )(adj, v, w, b.reshape(1, d))


def kernel(x, adj, W1, b1, W2, b2):
    x16 = x.astype(jnp.bfloat16)
    h16 = _gcn_layer(adj, x16, W1, b1, relu=True, out_dtype=jnp.bfloat16, bm=400)
    out = _gcn_layer(adj, h16, W2, b2, relu=False, out_dtype=jnp.float32, bm=400)
    return out
